# Initial kernel scaffold; baseline (speedup 1.0000x reference)
import jax
import jax.numpy as jnp
from jax.experimental import pallas as pl


def kernel(symbolic_instructions_batch, landmark_table, theta_table, radius_table):
    raise NotImplementedError("write your pallas kernel here")


# speedup vs baseline: 1.3255x; 1.3255x over previous
import functools

import jax
import jax.numpy as jnp
from jax import lax
from jax.experimental import pallas as pl
from jax.experimental.pallas import tpu as pltpu
from jax.experimental.pallas import tpu_sc as plsc

BATCH = 16384
EMBED = 64
NC = 2
NS = 16
NW = NC * NS
BPW = BATCH // NW
CHUNK = 128
NCH = BPW // CHUNK


def _sc_embed(idx0, idx1, landmark_table, theta_table):
  mesh = plsc.VectorSubcoreMesh(core_axis_name="c", subcore_axis_name="s")

  @functools.partial(
      pl.kernel,
      mesh=mesh,
      compiler_params=pltpu.CompilerParams(use_tc_tiling_on_sc=False),
      out_type=jax.ShapeDtypeStruct((BATCH, 2, EMBED), jnp.float32),
      scratch_types=[
          pltpu.VMEM((NCH, CHUNK), jnp.int32),
          pltpu.VMEM((NCH, CHUNK), jnp.int32),
          pltpu.VMEM((BPW, EMBED), jnp.float32),
          pltpu.VMEM((BPW, EMBED), jnp.float32),
          pltpu.SemaphoreType.DMA,
      ],
  )
  def body(idx0_hbm, idx1_hbm, lm_hbm, th_hbm, out_hbm,
           i0_v, i1_v, r0_v, r1_v, sem):
    wid = lax.axis_index("s") * NC + lax.axis_index("c")
    base = wid * BPW
    for j in range(NCH):
      pltpu.sync_copy(idx0_hbm.at[pl.ds(base + j * CHUNK, CHUNK)], i0_v.at[j])
      pltpu.sync_copy(idx1_hbm.at[pl.ds(base + j * CHUNK, CHUNK)], i1_v.at[j])
    copies = []
    for j in range(NCH):
      copies.append(pltpu.async_copy(
          lm_hbm.at[i0_v.at[j]], r0_v.at[pl.ds(j * CHUNK, CHUNK)], sem))
      copies.append(pltpu.async_copy(
          th_hbm.at[i1_v.at[j]], r1_v.at[pl.ds(j * CHUNK, CHUNK)], sem))
    for c in copies:
      c.wait()
    pltpu.sync_copy(r0_v, out_hbm.at[pl.ds(base, BPW), 0])
    pltpu.sync_copy(r1_v, out_hbm.at[pl.ds(base, BPW), 1])

  return body(idx0, idx1, landmark_table, theta_table)


def kernel(symbolic_instructions_batch, landmark_table, theta_table,
           radius_table):
  sib = symbolic_instructions_batch.astype(jnp.int32)
  idx0 = sib[:, 0]
  idx1 = sib[:, 1]
  out = _sc_embed(idx0, idx1, landmark_table, theta_table)
  return out.reshape(BATCH, 2 * EMBED)
